# baseline (speedup 1.0000x reference)
import functools

import jax
import jax.numpy as jnp
from jax import lax
from jax.experimental import pallas as pl
from jax.experimental.pallas import tpu as pltpu
from jax.experimental.pallas import tpu_sc as plsc

N = 10000
E = 320000
D = 128
DW = D // 2

NC = 2
NS = 16
NW = NC * NS
C = 64
NBUF = 3
SB = 2
G = 6
GRP = 6
CH = -(-E // (NW * C * GRP)) * GRP
NBLK = CH // G
EP = NW * CH * C
CP = (N // NS) // 8 * 8
REM = N - NS * CP
LANES = 16
MASK_HI = -65536


def _spmm_body(x_hbm, ed_hbm, wd_hbm, out_hbm,
               ed_v, wd_v, rows16_v, rows_f, dst_v, hi_sh,
               esem0, esem1, ssem0, ssem1, *gsems):
    esems = (esem0, esem1)
    ssems = (ssem0, ssem1)
    cid = lax.axis_index("c")
    sid = lax.axis_index("s")
    wid = cid * NS + sid

    def _zrow(r, _):
        for k in range(D // LANES):
            rows_f[0, r, pl.ds(k * LANES, LANES)] = jnp.zeros((LANES,), jnp.float32)
        return 0
    lax.fori_loop(0, C, _zrow, 0)
    base = sid * CP
    for off in range(0, CP, C):
        sz = min(C, CP - off)
        pltpu.sync_copy(rows_f.at[0].at[pl.ds(0, sz)],
                        hi_sh.at[pl.ds(base + off, sz)])

    @pl.when(sid == NS - 1)
    def _zero_rem():
        pltpu.sync_copy(rows_f.at[0].at[pl.ds(0, REM)],
                        hi_sh.at[pl.ds(NS * CP, REM)])
    plsc.subcore_barrier()

    pltpu.sync_copy(ed_hbm.at[wid, 0], ed_v.at[0])
    pltpu.sync_copy(wd_hbm.at[wid, 0], wd_v.at[0])
    pltpu.async_copy(ed_hbm.at[wid, 1], ed_v.at[1], esems[1])
    pltpu.async_copy(wd_hbm.at[wid, 1], wd_v.at[1], esems[1])
    for b in range(2):
        pltpu.async_copy(x_hbm.at[ed_v.at[0, 0, b]], rows16_v.at[b], gsems[b])

    def _group(g, _):
        for b in range(GRP):
            j = g * GRP + b
            gb = b % NBUF
            q = b % SB
            blk = j // G
            cj = lax.rem(j, G)
            p = lax.rem(blk, 2)
            pltpu.make_async_copy(x_hbm.at[ed_v.at[p, 0, cj]],
                                  rows16_v.at[gb], gsems[gb]).wait()

            if b < 2:
                @pl.when(g > 0)
                def _wait_sc():
                    pltpu.make_async_copy(rows_f.at[q], hi_sh.at[dst_v.at[q]],
                                          ssems[q]).wait()
            else:
                pltpu.make_async_copy(rows_f.at[q], hi_sh.at[dst_v.at[q]],
                                      ssems[q]).wait()

            def _scale(gr, _):
                wv = wd_v[p, cj, pl.ds(gr * LANES, LANES)]
                base_e = gr * LANES
                for i in range(LANES):
                    we = wv[i]
                    e = base_e + i
                    for k in range(DW // LANES):
                        v = rows16_v[gb, e, pl.ds(k * LANES, LANES)]
                        hi = lax.bitcast_convert_type(v & MASK_HI, jnp.float32)
                        lo = lax.bitcast_convert_type(v << 16, jnp.float32)
                        rows_f[q, e, pl.ds(k * LANES, LANES)] = hi * we
                        rows_f[q, e, pl.ds(DW + k * LANES, LANES)] = lo * we
                return 0
            lax.fori_loop(0, C // LANES, _scale, 0)

            for gr in range(C // LANES):
                lo_ = gr * LANES
                dst_v[q, pl.ds(lo_, LANES)] = ed_v[p, 1, cj, pl.ds(lo_, LANES)]

            pltpu.async_copy(rows_f.at[q], hi_sh.at[dst_v.at[q]],
                             ssems[q], add=True)

            refill = (cj == G - 1) & (blk < NBLK - 2)

            @pl.when(refill & (p == 0))
            def _refill0():
                pltpu.async_copy(ed_hbm.at[wid, blk + 2], ed_v.at[0], esems[0])
                pltpu.async_copy(wd_hbm.at[wid, blk + 2], wd_v.at[0], esems[0])

            @pl.when(refill & (p == 1))
            def _refill1():
                pltpu.async_copy(ed_hbm.at[wid, blk + 2], ed_v.at[1], esems[1])
                pltpu.async_copy(wd_hbm.at[wid, blk + 2], wd_v.at[1], esems[1])

            jn = j + 2
            blk2 = jn // G
            cj2 = lax.rem(jn, G)
            p2 = lax.rem(blk2, 2)
            newblk = (jn < CH) & (cj2 == 0)

            @pl.when(newblk & (p2 == 0))
            def _wait_ed0():
                pltpu.make_async_copy(ed_hbm.at[wid, blk2], ed_v.at[0],
                                      esems[0]).wait()
                pltpu.make_async_copy(wd_hbm.at[wid, blk2], wd_v.at[0],
                                      esems[0]).wait()

            @pl.when(newblk & (p2 == 1))
            def _wait_ed1():
                pltpu.make_async_copy(ed_hbm.at[wid, blk2], ed_v.at[1],
                                      esems[1]).wait()
                pltpu.make_async_copy(wd_hbm.at[wid, blk2], wd_v.at[1],
                                      esems[1]).wait()

            bn = (b + 2) % NBUF
            @pl.when(jn < CH)
            def _next_gather():
                pltpu.async_copy(x_hbm.at[ed_v.at[p2, 0, cj2]],
                                 rows16_v.at[bn], gsems[bn])
        return 0
    lax.fori_loop(0, CH // GRP, _group, 0)

    pltpu.make_async_copy(rows_f.at[0], hi_sh.at[dst_v.at[0]], ssems[0]).wait()
    pltpu.make_async_copy(rows_f.at[1], hi_sh.at[dst_v.at[1]], ssems[1]).wait()

    plsc.subcore_barrier()
    pltpu.sync_copy(hi_sh.at[pl.ds(base, CP)],
                    out_hbm.at[cid].at[pl.ds(base, CP)])

    @pl.when(sid == NS - 1)
    def _copy_rem():
        pltpu.sync_copy(hi_sh.at[pl.ds(NS * CP, REM)],
                        out_hbm.at[cid].at[pl.ds(NS * CP, REM)])


_spmm = functools.partial(
    pl.kernel,
    out_type=jax.ShapeDtypeStruct((NC, N, D), jnp.float32),
    mesh=plsc.VectorSubcoreMesh(core_axis_name="c", subcore_axis_name="s"),
    compiler_params=pltpu.CompilerParams(use_tc_tiling_on_sc=False),
    scratch_types=[
        pltpu.VMEM((2, 2, G, C), jnp.int32),
        pltpu.VMEM((2, G, C), jnp.float32),
        pltpu.VMEM((NBUF, C, DW), jnp.int32),
        pltpu.VMEM((SB, C, D), jnp.float32),
        pltpu.VMEM((SB, C), jnp.int32),
        pltpu.VMEM_SHARED((N, D), jnp.float32),
        pltpu.SemaphoreType.DMA,
        pltpu.SemaphoreType.DMA,
        pltpu.SemaphoreType.DMA,
        pltpu.SemaphoreType.DMA,
    ] + [pltpu.SemaphoreType.DMA] * NBUF,
)(_spmm_body)


BN = 2000


def _combine_body(scal_ref, hi_ref, h0_ref, w_ref, out_ref):
    a = scal_ref[0]
    b = scal_ref[1]
    th = scal_ref[2]
    sup = a * (hi_ref[0] + hi_ref[1]) + b * h0_ref[...]
    r = lax.broadcasted_iota(jnp.int32, (D, D), 0)
    c = lax.broadcasted_iota(jnp.int32, (D, D), 1)
    eye = jnp.where(r == c, jnp.float32(1), jnp.float32(0))
    weff = th * w_ref[...] + (jnp.float32(1) - th) * eye
    out_ref[...] = jnp.dot(sup, weff, preferred_element_type=jnp.float32)


def _combine(scal, hi2, h0, weight):
    return pl.pallas_call(
        _combine_body,
        grid=(N // BN,),
        in_specs=[
            pl.BlockSpec(memory_space=pltpu.SMEM),
            pl.BlockSpec((NC, BN, D), lambda i: (0, i, 0)),
            pl.BlockSpec((BN, D), lambda i: (i, 0)),
            pl.BlockSpec((D, D), lambda i: (0, 0)),
        ],
        out_specs=pl.BlockSpec((BN, D), lambda i: (i, 0)),
        out_shape=jax.ShapeDtypeStruct((N, D), jnp.float32),
    )(scal, hi2, h0, weight)


def kernel(input, adj_edge_weight, h0, weight, adj_edge_index, lamda, alpha, l):
    theta = jnp.log(jnp.float32(lamda) / l + 1).astype(jnp.float32)
    alpha = jnp.float32(alpha)
    dst = adj_edge_index[0]
    src = adj_edge_index[1]

    xb = input.astype(jnp.bfloat16)
    hi_b = lax.bitcast_convert_type(xb[:, :DW], jnp.uint16).astype(jnp.uint32)
    lo_b = lax.bitcast_convert_type(xb[:, DW:], jnp.uint16).astype(jnp.uint32)
    x_pack = lax.bitcast_convert_type((hi_b << 16) | lo_b, jnp.int32)

    pad = EP - E
    pad_idx = (jnp.arange(pad, dtype=jnp.int32) * 8) % N
    src_p = jnp.concatenate([src, pad_idx]).reshape(NW, NBLK, G, C)
    dst_p = jnp.concatenate([dst, pad_idx]).reshape(NW, NBLK, G, C)
    w_p = jnp.concatenate(
        [adj_edge_weight, jnp.zeros((pad,), jnp.float32)]).reshape(NW, NBLK, G, C)
    ed = jnp.stack([src_p, dst_p], axis=2)

    hi2 = _spmm(x_pack, ed, w_p)

    scal = jnp.stack([jnp.float32(1) - alpha, alpha, theta])
    return _combine(scal, hi2, h0, weight)


# speedup vs baseline: 1.9773x; 1.9773x over previous
import functools

import jax
import jax.numpy as jnp
from jax import lax
from jax.experimental import pallas as pl
from jax.experimental.pallas import tpu as pltpu
from jax.experimental.pallas import tpu_sc as plsc

N = 10000
E = 320000
D = 128

NC = 2
NS = 16
NW = NC * NS
C = 120
NBUF = 3
G = 4
GRP = NBUF * G
CH = -(-E // (NW * C * GRP)) * GRP
NBLK = CH // G
EP = NW * CH * C
CP = (N // NS) // 8 * 8
REM = N - NS * CP
LANES = 16


def _spmm_body(x_hbm, ed_hbm, wd_hbm, out_hbm,
               ed_v, wd_v, rows_v, dst_v, hi_sh,
               esem0, esem1, ssem0, ssem1, ssem2, *gsems):
    esems = (esem0, esem1)
    ssems = (ssem0, ssem1, ssem2)
    cid = lax.axis_index("c")
    sid = lax.axis_index("s")
    wid = cid * NS + sid

    def _zrow(r, _):
        for k in range(D // LANES):
            rows_v[0, r, pl.ds(k * LANES, LANES)] = jnp.zeros((LANES,), jnp.float32)
        return 0
    lax.fori_loop(0, C, _zrow, 0)
    base = sid * CP
    for off in range(0, CP, C):
        sz = min(C, CP - off)
        pltpu.sync_copy(rows_v.at[0].at[pl.ds(0, sz)],
                        hi_sh.at[pl.ds(base + off, sz)])

    @pl.when(sid == NS - 1)
    def _zero_rem():
        pltpu.sync_copy(rows_v.at[0].at[pl.ds(0, REM)],
                        hi_sh.at[pl.ds(NS * CP, REM)])
    plsc.subcore_barrier()

    pltpu.sync_copy(ed_hbm.at[wid, 0], ed_v.at[0])
    pltpu.sync_copy(wd_hbm.at[wid, 0], wd_v.at[0])
    pltpu.async_copy(ed_hbm.at[wid, 1], ed_v.at[1], esems[1])
    pltpu.async_copy(wd_hbm.at[wid, 1], wd_v.at[1], esems[1])
    H = 64
    for b in range(2):
        pltpu.async_copy(x_hbm.at[ed_v.at[0, 0, b].at[pl.ds(0, H)]],
                         rows_v.at[b].at[pl.ds(0, H)], gsems[b])
        pltpu.async_copy(x_hbm.at[ed_v.at[0, 0, b].at[pl.ds(H, C - H)]],
                         rows_v.at[b].at[pl.ds(H, C - H)], gsems[b])

    def _group(g, _):
        for b in range(NBUF):
            j = g * NBUF + b
            blk = j // G
            cj = lax.rem(j, G)
            p = lax.rem(blk, 2)
            pltpu.make_async_copy(x_hbm.at[ed_v.at[p, 0, cj].at[pl.ds(0, H)]],
                                  rows_v.at[b].at[pl.ds(0, H)], gsems[b]).wait()
            pltpu.make_async_copy(x_hbm.at[ed_v.at[p, 0, cj].at[pl.ds(H, C - H)]],
                                  rows_v.at[b].at[pl.ds(H, C - H)], gsems[b]).wait()

            def _scale(gr, _):
                wv = wd_v[p, cj, pl.ds(gr * LANES, LANES)]
                base_e = gr * LANES
                for i in range(LANES):
                    we = wv[i]
                    for k in range(D // LANES):
                        sl = pl.ds(k * LANES, LANES)
                        rows_v[b, base_e + i, sl] = rows_v[b, base_e + i, sl] * we
                return 0
            lax.fori_loop(0, C // LANES, _scale, 0)
            if C % LANES:
                wvt = wd_v[p, cj, pl.ds(C - LANES, LANES)]
                for i in range(C // LANES * LANES, C):
                    we = wvt[i - (C - LANES)]
                    for k in range(D // LANES):
                        sl = pl.ds(k * LANES, LANES)
                        rows_v[b, i, sl] = rows_v[b, i, sl] * we

            for gr in range(C // LANES):
                lo = gr * LANES
                dst_v[b, pl.ds(lo, LANES)] = ed_v[p, 1, cj, pl.ds(lo, LANES)]
            if C % LANES:
                lo = C - LANES
                dst_v[b, pl.ds(lo, LANES)] = ed_v[p, 1, cj, pl.ds(lo, LANES)]

            pltpu.async_copy(rows_v.at[b], hi_sh.at[dst_v.at[b]],
                             ssems[b], add=True)

            bp = (b - 1) % NBUF
            if b == 0:
                @pl.when(g > 0)
                def _wait_sc():
                    pltpu.make_async_copy(rows_v.at[bp], hi_sh.at[dst_v.at[bp]],
                                          ssems[bp]).wait()
            else:
                pltpu.make_async_copy(rows_v.at[bp], hi_sh.at[dst_v.at[bp]],
                                      ssems[bp]).wait()

            refill = (cj == G - 1) & (blk < NBLK - 2)

            @pl.when(refill & (p == 0))
            def _refill0():
                pltpu.async_copy(ed_hbm.at[wid, blk + 2], ed_v.at[0], esems[0])
                pltpu.async_copy(wd_hbm.at[wid, blk + 2], wd_v.at[0], esems[0])

            @pl.when(refill & (p == 1))
            def _refill1():
                pltpu.async_copy(ed_hbm.at[wid, blk + 2], ed_v.at[1], esems[1])
                pltpu.async_copy(wd_hbm.at[wid, blk + 2], wd_v.at[1], esems[1])

            jn = j + 2
            blk2 = jn // G
            cj2 = lax.rem(jn, G)
            p2 = lax.rem(blk2, 2)
            newblk = (jn < CH) & (cj2 == 0)

            @pl.when(newblk & (p2 == 0))
            def _wait_ed0():
                pltpu.make_async_copy(ed_hbm.at[wid, blk2], ed_v.at[0],
                                      esems[0]).wait()
                pltpu.make_async_copy(wd_hbm.at[wid, blk2], wd_v.at[0],
                                      esems[0]).wait()

            @pl.when(newblk & (p2 == 1))
            def _wait_ed1():
                pltpu.make_async_copy(ed_hbm.at[wid, blk2], ed_v.at[1],
                                      esems[1]).wait()
                pltpu.make_async_copy(wd_hbm.at[wid, blk2], wd_v.at[1],
                                      esems[1]).wait()

            bn = (b + 2) % NBUF
            @pl.when(jn < CH)
            def _next_gather():
                pltpu.async_copy(x_hbm.at[ed_v.at[p2, 0, cj2].at[pl.ds(0, H)]],
                                 rows_v.at[bn].at[pl.ds(0, H)], gsems[bn])
                pltpu.async_copy(x_hbm.at[ed_v.at[p2, 0, cj2].at[pl.ds(H, C - H)]],
                                 rows_v.at[bn].at[pl.ds(H, C - H)], gsems[bn])
        return 0
    lax.fori_loop(0, CH // NBUF, _group, 0)

    pltpu.make_async_copy(rows_v.at[(CH - 1) % NBUF],
                          hi_sh.at[dst_v.at[(CH - 1) % NBUF]],
                          ssems[(CH - 1) % NBUF]).wait()

    plsc.subcore_barrier()
    pltpu.sync_copy(hi_sh.at[pl.ds(base, CP)],
                    out_hbm.at[cid].at[pl.ds(base, CP)])

    @pl.when(sid == NS - 1)
    def _copy_rem():
        pltpu.sync_copy(hi_sh.at[pl.ds(NS * CP, REM)],
                        out_hbm.at[cid].at[pl.ds(NS * CP, REM)])


_spmm = functools.partial(
    pl.kernel,
    out_type=jax.ShapeDtypeStruct((NC, N, D), jnp.float32),
    mesh=plsc.VectorSubcoreMesh(core_axis_name="c", subcore_axis_name="s"),
    scratch_types=[
        pltpu.VMEM((2, 2, G, C), jnp.int32),
        pltpu.VMEM((2, G, C), jnp.float32),
        pltpu.VMEM((NBUF, C, D), jnp.float32),
        pltpu.VMEM((NBUF, C), jnp.int32),
        pltpu.VMEM_SHARED((N, D), jnp.float32),
        pltpu.SemaphoreType.DMA,
        pltpu.SemaphoreType.DMA,
    ] + [pltpu.SemaphoreType.DMA] * (2 * NBUF),
)(_spmm_body)


BN = 2000


def _combine_body(scal_ref, hi_ref, h0_ref, w_ref, out_ref):
    a = scal_ref[0]
    b = scal_ref[1]
    th = scal_ref[2]
    sup = a * (hi_ref[0] + hi_ref[1]) + b * h0_ref[...]
    r = lax.broadcasted_iota(jnp.int32, (D, D), 0)
    c = lax.broadcasted_iota(jnp.int32, (D, D), 1)
    eye = jnp.where(r == c, jnp.float32(1), jnp.float32(0))
    weff = th * w_ref[...] + (jnp.float32(1) - th) * eye
    out_ref[...] = jnp.dot(sup, weff, preferred_element_type=jnp.float32)


def _combine(scal, hi2, h0, weight):
    return pl.pallas_call(
        _combine_body,
        grid=(N // BN,),
        in_specs=[
            pl.BlockSpec(memory_space=pltpu.SMEM),
            pl.BlockSpec((NC, BN, D), lambda i: (0, i, 0)),
            pl.BlockSpec((BN, D), lambda i: (i, 0)),
            pl.BlockSpec((D, D), lambda i: (0, 0)),
        ],
        out_specs=pl.BlockSpec((BN, D), lambda i: (i, 0)),
        out_shape=jax.ShapeDtypeStruct((N, D), jnp.float32),
    )(scal, hi2, h0, weight)


def kernel(input, adj_edge_weight, h0, weight, adj_edge_index, lamda, alpha, l):
    theta = jnp.log(jnp.float32(lamda) / l + 1).astype(jnp.float32)
    alpha = jnp.float32(alpha)
    dst = adj_edge_index[0]
    src = adj_edge_index[1]

    pad = EP - E
    pad_idx = (jnp.arange(pad, dtype=jnp.int32) * 8) % N
    src_p = jnp.concatenate([src, pad_idx]).reshape(NW, NBLK, G, C)
    dst_p = jnp.concatenate([dst, pad_idx]).reshape(NW, NBLK, G, C)
    w_p = jnp.concatenate(
        [adj_edge_weight, jnp.zeros((pad,), jnp.float32)]).reshape(NW, NBLK, G, C)
    ed = jnp.stack([src_p, dst_p], axis=2)

    hi2 = _spmm(input, ed, w_p)

    scal = jnp.stack([jnp.float32(1) - alpha, alpha, theta])
    return _combine(scal, hi2, h0, weight)


# speedup vs baseline: 1.9873x; 1.0051x over previous
import functools

import jax
import jax.numpy as jnp
from jax import lax
from jax.experimental import pallas as pl
from jax.experimental.pallas import tpu as pltpu
from jax.experimental.pallas import tpu_sc as plsc

N = 10000
E = 320000
D = 128

NC = 2
NS = 16
NW = NC * NS
C = 120
NBUF = 3
G = 4
GRP = NBUF * G
CH = -(-E // (NW * C * GRP)) * GRP
NBLK = CH // G
EP = NW * CH * C
CP = (N // NS) // 8 * 8
REM = N - NS * CP
LANES = 16


def _spmm_body(x_hbm, sd_hbm, dd_hbm, wd_hbm, out_hbm,
               sd_v, dd_v, wd_v, rows_v, dst_v, hi_sh,
               esem0, esem1, ssem0, ssem1, ssem2, *gsems):
    esems = (esem0, esem1)
    ssems = (ssem0, ssem1, ssem2)
    cid = lax.axis_index("c")
    sid = lax.axis_index("s")
    wid = cid * NS + sid

    pltpu.sync_copy(sd_hbm.at[wid, 0], sd_v.at[0])
    pltpu.sync_copy(dd_hbm.at[wid, 0], dd_v.at[0])
    pltpu.sync_copy(wd_hbm.at[wid, 0], wd_v.at[0])
    pltpu.async_copy(sd_hbm.at[wid, 1], sd_v.at[1], esems[1])
    pltpu.async_copy(dd_hbm.at[wid, 1], dd_v.at[1], esems[1])
    pltpu.async_copy(wd_hbm.at[wid, 1], wd_v.at[1], esems[1])

    def _zrow(r, _):
        for k in range(D // LANES):
            rows_v[2, r, pl.ds(k * LANES, LANES)] = jnp.zeros((LANES,), jnp.float32)
        return 0
    lax.fori_loop(0, C, _zrow, 0)

    H = 64
    for b in range(2):
        pltpu.async_copy(x_hbm.at[sd_v.at[0, b].at[pl.ds(0, H)]],
                         rows_v.at[b].at[pl.ds(0, H)], gsems[b])
        pltpu.async_copy(x_hbm.at[sd_v.at[0, b].at[pl.ds(H, C - H)]],
                         rows_v.at[b].at[pl.ds(H, C - H)], gsems[b])

    base = sid * CP
    for off in range(0, CP, C):
        sz = min(C, CP - off)
        pltpu.sync_copy(rows_v.at[2].at[pl.ds(0, sz)],
                        hi_sh.at[pl.ds(base + off, sz)])

    @pl.when(sid == NS - 1)
    def _zero_rem():
        pltpu.sync_copy(rows_v.at[2].at[pl.ds(0, REM)],
                        hi_sh.at[pl.ds(NS * CP, REM)])
    plsc.subcore_barrier()

    def _group(g, _):
        for b in range(NBUF):
            j = g * NBUF + b
            blk = j // G
            cj = lax.rem(j, G)
            p = lax.rem(blk, 2)
            pltpu.make_async_copy(x_hbm.at[sd_v.at[p, cj].at[pl.ds(0, H)]],
                                  rows_v.at[b].at[pl.ds(0, H)], gsems[b]).wait()
            pltpu.make_async_copy(x_hbm.at[sd_v.at[p, cj].at[pl.ds(H, C - H)]],
                                  rows_v.at[b].at[pl.ds(H, C - H)], gsems[b]).wait()

            def _scale(gr, _):
                wv = wd_v[p, cj, pl.ds(gr * LANES, LANES)]
                base_e = gr * LANES
                for i in range(LANES):
                    we = wv[i]
                    for k in range(D // LANES):
                        sl = pl.ds(k * LANES, LANES)
                        rows_v[b, base_e + i, sl] = rows_v[b, base_e + i, sl] * we
                return 0
            lax.fori_loop(0, C // LANES, _scale, 0)
            if C % LANES:
                wvt = wd_v[p, cj, pl.ds(C - LANES, LANES)]
                for i in range(C // LANES * LANES, C):
                    we = wvt[i - (C - LANES)]
                    for k in range(D // LANES):
                        sl = pl.ds(k * LANES, LANES)
                        rows_v[b, i, sl] = rows_v[b, i, sl] * we

            for gr in range(C // LANES):
                lo = gr * LANES
                dst_v[b, pl.ds(lo, LANES)] = dd_v[p, cj, pl.ds(lo, LANES)]
            if C % LANES:
                lo = C - LANES
                dst_v[b, pl.ds(lo, LANES)] = dd_v[p, cj, pl.ds(lo, LANES)]

            pltpu.async_copy(rows_v.at[b], hi_sh.at[dst_v.at[b]],
                             ssems[b], add=True)

            bp = (b - 1) % NBUF
            if b == 0:
                @pl.when(g > 0)
                def _wait_sc():
                    pltpu.make_async_copy(rows_v.at[bp], hi_sh.at[dst_v.at[bp]],
                                          ssems[bp]).wait()
            else:
                pltpu.make_async_copy(rows_v.at[bp], hi_sh.at[dst_v.at[bp]],
                                      ssems[bp]).wait()

            refill = (cj == G - 1) & (blk < NBLK - 2)

            @pl.when(refill & (p == 0))
            def _refill0():
                pltpu.async_copy(sd_hbm.at[wid, blk + 2], sd_v.at[0], esems[0])
                pltpu.async_copy(dd_hbm.at[wid, blk + 2], dd_v.at[0], esems[0])
                pltpu.async_copy(wd_hbm.at[wid, blk + 2], wd_v.at[0], esems[0])

            @pl.when(refill & (p == 1))
            def _refill1():
                pltpu.async_copy(sd_hbm.at[wid, blk + 2], sd_v.at[1], esems[1])
                pltpu.async_copy(dd_hbm.at[wid, blk + 2], dd_v.at[1], esems[1])
                pltpu.async_copy(wd_hbm.at[wid, blk + 2], wd_v.at[1], esems[1])

            jn = j + 2
            blk2 = jn // G
            cj2 = lax.rem(jn, G)
            p2 = lax.rem(blk2, 2)
            newblk = (jn < CH) & (cj2 == 0)

            @pl.when(newblk & (p2 == 0))
            def _wait_ed0():
                pltpu.make_async_copy(sd_hbm.at[wid, blk2], sd_v.at[0],
                                      esems[0]).wait()
                pltpu.make_async_copy(dd_hbm.at[wid, blk2], dd_v.at[0],
                                      esems[0]).wait()
                pltpu.make_async_copy(wd_hbm.at[wid, blk2], wd_v.at[0],
                                      esems[0]).wait()

            @pl.when(newblk & (p2 == 1))
            def _wait_ed1():
                pltpu.make_async_copy(sd_hbm.at[wid, blk2], sd_v.at[1],
                                      esems[1]).wait()
                pltpu.make_async_copy(dd_hbm.at[wid, blk2], dd_v.at[1],
                                      esems[1]).wait()
                pltpu.make_async_copy(wd_hbm.at[wid, blk2], wd_v.at[1],
                                      esems[1]).wait()

            bn = (b + 2) % NBUF
            @pl.when(jn < CH)
            def _next_gather():
                pltpu.async_copy(x_hbm.at[sd_v.at[p2, cj2].at[pl.ds(0, H)]],
                                 rows_v.at[bn].at[pl.ds(0, H)], gsems[bn])
                pltpu.async_copy(x_hbm.at[sd_v.at[p2, cj2].at[pl.ds(H, C - H)]],
                                 rows_v.at[bn].at[pl.ds(H, C - H)], gsems[bn])
        return 0
    lax.fori_loop(0, CH // NBUF, _group, 0)

    pltpu.make_async_copy(rows_v.at[(CH - 1) % NBUF],
                          hi_sh.at[dst_v.at[(CH - 1) % NBUF]],
                          ssems[(CH - 1) % NBUF]).wait()

    plsc.subcore_barrier()
    pltpu.sync_copy(hi_sh.at[pl.ds(base, CP)],
                    out_hbm.at[cid].at[pl.ds(base, CP)])

    @pl.when(sid == NS - 1)
    def _copy_rem():
        pltpu.sync_copy(hi_sh.at[pl.ds(NS * CP, REM)],
                        out_hbm.at[cid].at[pl.ds(NS * CP, REM)])


_spmm = functools.partial(
    pl.kernel,
    out_type=jax.ShapeDtypeStruct((NC, N, D), jnp.float32),
    mesh=plsc.VectorSubcoreMesh(core_axis_name="c", subcore_axis_name="s"),
    scratch_types=[
        pltpu.VMEM((2, G, C), jnp.int32),
        pltpu.VMEM((2, G, C), jnp.int32),
        pltpu.VMEM((2, G, C), jnp.float32),
        pltpu.VMEM((NBUF, C, D), jnp.float32),
        pltpu.VMEM((NBUF, C), jnp.int32),
        pltpu.VMEM_SHARED((N, D), jnp.float32),
        pltpu.SemaphoreType.DMA,
        pltpu.SemaphoreType.DMA,
    ] + [pltpu.SemaphoreType.DMA] * (2 * NBUF),
)(_spmm_body)


BN = 2000


def _combine_body(scal_ref, hi_ref, h0_ref, w_ref, out_ref):
    a = scal_ref[0]
    b = scal_ref[1]
    th = scal_ref[2]
    sup = a * (hi_ref[0] + hi_ref[1]) + b * h0_ref[...]
    r = lax.broadcasted_iota(jnp.int32, (D, D), 0)
    c = lax.broadcasted_iota(jnp.int32, (D, D), 1)
    eye = jnp.where(r == c, jnp.float32(1), jnp.float32(0))
    weff = th * w_ref[...] + (jnp.float32(1) - th) * eye
    out_ref[...] = jnp.dot(sup, weff, preferred_element_type=jnp.float32)


def _combine(scal, hi2, h0, weight):
    return pl.pallas_call(
        _combine_body,
        grid=(N // BN,),
        in_specs=[
            pl.BlockSpec(memory_space=pltpu.SMEM),
            pl.BlockSpec((NC, BN, D), lambda i: (0, i, 0)),
            pl.BlockSpec((BN, D), lambda i: (i, 0)),
            pl.BlockSpec((D, D), lambda i: (0, 0)),
        ],
        out_specs=pl.BlockSpec((BN, D), lambda i: (i, 0)),
        out_shape=jax.ShapeDtypeStruct((N, D), jnp.float32),
    )(scal, hi2, h0, weight)


def kernel(input, adj_edge_weight, h0, weight, adj_edge_index, lamda, alpha, l):
    theta = jnp.log(jnp.float32(lamda) / l + 1).astype(jnp.float32)
    alpha = jnp.float32(alpha)
    dst = adj_edge_index[0]
    src = adj_edge_index[1]

    pad = EP - E
    pad_idx = (jnp.arange(pad, dtype=jnp.int32) * 8) % N
    src_p = jnp.concatenate([src, pad_idx]).reshape(NW, NBLK, G, C)
    dst_p = jnp.concatenate([dst, pad_idx]).reshape(NW, NBLK, G, C)
    w_p = jnp.concatenate(
        [adj_edge_weight, jnp.zeros((pad,), jnp.float32)]).reshape(NW, NBLK, G, C)

    hi2 = _spmm(input, src_p, dst_p, w_p)

    scal = jnp.stack([jnp.float32(1) - alpha, alpha, theta])
    return _combine(scal, hi2, h0, weight)


# speedup vs baseline: 1.9999x; 1.0064x over previous
import functools

import jax
import jax.numpy as jnp
from jax import lax
from jax.experimental import pallas as pl
from jax.experimental.pallas import tpu as pltpu
from jax.experimental.pallas import tpu_sc as plsc

N = 10000
E = 320000
D = 128

NC = 2
NS = 16
NW = NC * NS
C = 120
NBUF = 3
G = 4
GRP = NBUF * G
CH = -(-E // (NW * C * GRP)) * GRP
NBLK = CH // G
EP = NW * CH * C
CP = (N // NS) // 8 * 8
REM = N - NS * CP
LANES = 16


def _spmm_body(x_hbm, sd_hbm, dd_hbm, wd_hbm, out_hbm,
               sd_v, dd_v, wd_v, rows_v, dst_v, hi_sh,
               esem0, esem1, ssem0, ssem1, ssem2, *gsems):
    esems = (esem0, esem1)
    ssems = (ssem0, ssem1, ssem2)
    cid = lax.axis_index("c")
    sid = lax.axis_index("s")
    wid = cid * NS + sid

    pltpu.sync_copy(sd_hbm.at[wid, 0], sd_v.at[0])
    pltpu.sync_copy(dd_hbm.at[wid, 0], dd_v.at[0])
    pltpu.sync_copy(wd_hbm.at[wid, 0], wd_v.at[0])
    pltpu.async_copy(sd_hbm.at[wid, 1], sd_v.at[1], esems[1])
    pltpu.async_copy(dd_hbm.at[wid, 1], dd_v.at[1], esems[1])
    pltpu.async_copy(wd_hbm.at[wid, 1], wd_v.at[1], esems[1])

    def _zrow(r, _):
        for k in range(D // LANES):
            rows_v[2, r, pl.ds(k * LANES, LANES)] = jnp.zeros((LANES,), jnp.float32)
        return 0
    lax.fori_loop(0, C, _zrow, 0)

    H = 64
    for b in range(2):
        pltpu.async_copy(x_hbm.at[sd_v.at[0, b].at[pl.ds(0, H)]],
                         rows_v.at[b].at[pl.ds(0, H)], gsems[b])
        pltpu.async_copy(x_hbm.at[sd_v.at[0, b].at[pl.ds(H, C - H)]],
                         rows_v.at[b].at[pl.ds(H, C - H)], gsems[b])

    base = sid * CP
    for off in range(0, CP, C):
        sz = min(C, CP - off)
        pltpu.sync_copy(rows_v.at[2].at[pl.ds(0, sz)],
                        hi_sh.at[pl.ds(base + off, sz)])

    @pl.when(sid == NS - 1)
    def _zero_rem():
        pltpu.sync_copy(rows_v.at[2].at[pl.ds(0, REM)],
                        hi_sh.at[pl.ds(NS * CP, REM)])
    plsc.subcore_barrier()

    def _group(g, _):
        for b in range(NBUF):
            j = g * NBUF + b
            blk = j // G
            cj = lax.rem(j, G)
            p = lax.rem(blk, 2)
            pltpu.make_async_copy(x_hbm.at[sd_v.at[p, cj].at[pl.ds(0, H)]],
                                  rows_v.at[b].at[pl.ds(0, H)], gsems[b]).wait()
            pltpu.make_async_copy(x_hbm.at[sd_v.at[p, cj].at[pl.ds(H, C - H)]],
                                  rows_v.at[b].at[pl.ds(H, C - H)], gsems[b]).wait()

            def _scale(gr, _):
                wv = wd_v[p, cj, pl.ds(gr * LANES, LANES)]
                base_e = gr * LANES
                for i in range(LANES):
                    we = wv[i]
                    for k in range(D // LANES):
                        sl = pl.ds(k * LANES, LANES)
                        rows_v[b, base_e + i, sl] = rows_v[b, base_e + i, sl] * we
                return 0
            lax.fori_loop(0, C // LANES, _scale, 0)
            if C % LANES:
                wvt = wd_v[p, cj, pl.ds(C - LANES, LANES)]
                for i in range(C // LANES * LANES, C):
                    we = wvt[i - (C - LANES)]
                    for k in range(D // LANES):
                        sl = pl.ds(k * LANES, LANES)
                        rows_v[b, i, sl] = rows_v[b, i, sl] * we

            for gr in range(C // LANES):
                lo = gr * LANES
                dst_v[b, pl.ds(lo, LANES)] = dd_v[p, cj, pl.ds(lo, LANES)]
            if C % LANES:
                lo = C - LANES
                dst_v[b, pl.ds(lo, LANES)] = dd_v[p, cj, pl.ds(lo, LANES)]

            pltpu.async_copy(rows_v.at[b], hi_sh.at[dst_v.at[b]],
                             ssems[b], add=True)

            bp = (b - 1) % NBUF
            if b == 0:
                @pl.when(g > 0)
                def _wait_sc():
                    pltpu.make_async_copy(rows_v.at[bp], hi_sh.at[dst_v.at[bp]],
                                          ssems[bp]).wait()
            else:
                pltpu.make_async_copy(rows_v.at[bp], hi_sh.at[dst_v.at[bp]],
                                      ssems[bp]).wait()

            refill = (cj == G - 1) & (blk < NBLK - 2)

            @pl.when(refill & (p == 0))
            def _refill0():
                pltpu.async_copy(sd_hbm.at[wid, blk + 2], sd_v.at[0], esems[0])
                pltpu.async_copy(dd_hbm.at[wid, blk + 2], dd_v.at[0], esems[0])
                pltpu.async_copy(wd_hbm.at[wid, blk + 2], wd_v.at[0], esems[0])

            @pl.when(refill & (p == 1))
            def _refill1():
                pltpu.async_copy(sd_hbm.at[wid, blk + 2], sd_v.at[1], esems[1])
                pltpu.async_copy(dd_hbm.at[wid, blk + 2], dd_v.at[1], esems[1])
                pltpu.async_copy(wd_hbm.at[wid, blk + 2], wd_v.at[1], esems[1])

            jn = j + 2
            blk2 = jn // G
            cj2 = lax.rem(jn, G)
            p2 = lax.rem(blk2, 2)
            newblk = (jn < CH) & (cj2 == 0)

            @pl.when(newblk & (p2 == 0))
            def _wait_ed0():
                pltpu.make_async_copy(sd_hbm.at[wid, blk2], sd_v.at[0],
                                      esems[0]).wait()
                pltpu.make_async_copy(dd_hbm.at[wid, blk2], dd_v.at[0],
                                      esems[0]).wait()
                pltpu.make_async_copy(wd_hbm.at[wid, blk2], wd_v.at[0],
                                      esems[0]).wait()

            @pl.when(newblk & (p2 == 1))
            def _wait_ed1():
                pltpu.make_async_copy(sd_hbm.at[wid, blk2], sd_v.at[1],
                                      esems[1]).wait()
                pltpu.make_async_copy(dd_hbm.at[wid, blk2], dd_v.at[1],
                                      esems[1]).wait()
                pltpu.make_async_copy(wd_hbm.at[wid, blk2], wd_v.at[1],
                                      esems[1]).wait()

            bn = (b + 2) % NBUF
            @pl.when(jn < CH)
            def _next_gather():
                pltpu.async_copy(x_hbm.at[sd_v.at[p2, cj2].at[pl.ds(0, H)]],
                                 rows_v.at[bn].at[pl.ds(0, H)], gsems[bn])
                pltpu.async_copy(x_hbm.at[sd_v.at[p2, cj2].at[pl.ds(H, C - H)]],
                                 rows_v.at[bn].at[pl.ds(H, C - H)], gsems[bn])
        return 0
    lax.fori_loop(0, CH // NBUF, _group, 0)

    pltpu.make_async_copy(rows_v.at[(CH - 1) % NBUF],
                          hi_sh.at[dst_v.at[(CH - 1) % NBUF]],
                          ssems[(CH - 1) % NBUF]).wait()

    plsc.subcore_barrier()
    pltpu.sync_copy(hi_sh.at[pl.ds(base, CP)],
                    out_hbm.at[cid].at[pl.ds(base, CP)])

    @pl.when(sid == NS - 1)
    def _copy_rem():
        pltpu.sync_copy(hi_sh.at[pl.ds(NS * CP, REM)],
                        out_hbm.at[cid].at[pl.ds(NS * CP, REM)])


_spmm = functools.partial(
    pl.kernel,
    out_type=jax.ShapeDtypeStruct((NC, N, D), jnp.float32),
    mesh=plsc.VectorSubcoreMesh(core_axis_name="c", subcore_axis_name="s"),
    scratch_types=[
        pltpu.VMEM((2, G, C), jnp.int32),
        pltpu.VMEM((2, G, C), jnp.int32),
        pltpu.VMEM((2, G, C), jnp.float32),
        pltpu.VMEM((NBUF, C, D), jnp.float32),
        pltpu.VMEM((NBUF, C), jnp.int32),
        pltpu.VMEM_SHARED((N, D), jnp.float32),
        pltpu.SemaphoreType.DMA,
        pltpu.SemaphoreType.DMA,
    ] + [pltpu.SemaphoreType.DMA] * (2 * NBUF),
)(_spmm_body)


BN = 5000


def _combine_body(scal_ref, hi_ref, h0_ref, w_ref, out_ref):
    a = scal_ref[0]
    b = scal_ref[1]
    th = scal_ref[2]
    sup = a * (hi_ref[0] + hi_ref[1]) + b * h0_ref[...]
    r = lax.broadcasted_iota(jnp.int32, (D, D), 0)
    c = lax.broadcasted_iota(jnp.int32, (D, D), 1)
    eye = jnp.where(r == c, jnp.float32(1), jnp.float32(0))
    weff = th * w_ref[...] + (jnp.float32(1) - th) * eye
    out_ref[...] = jnp.dot(sup, weff, preferred_element_type=jnp.float32)


def _combine(scal, hi2, h0, weight):
    return pl.pallas_call(
        _combine_body,
        grid=(N // BN,),
        in_specs=[
            pl.BlockSpec(memory_space=pltpu.SMEM),
            pl.BlockSpec((NC, BN, D), lambda i: (0, i, 0)),
            pl.BlockSpec((BN, D), lambda i: (i, 0)),
            pl.BlockSpec((D, D), lambda i: (0, 0)),
        ],
        out_specs=pl.BlockSpec((BN, D), lambda i: (i, 0)),
        out_shape=jax.ShapeDtypeStruct((N, D), jnp.float32),
    )(scal, hi2, h0, weight)


def kernel(input, adj_edge_weight, h0, weight, adj_edge_index, lamda, alpha, l):
    theta = jnp.log(jnp.float32(lamda) / l + 1).astype(jnp.float32)
    alpha = jnp.float32(alpha)
    dst = adj_edge_index[0]
    src = adj_edge_index[1]

    pad = EP - E
    pad_idx = (jnp.arange(pad, dtype=jnp.int32) * 8) % N
    src_p = jnp.concatenate([src, pad_idx]).reshape(NW, NBLK, G, C)
    dst_p = jnp.concatenate([dst, pad_idx]).reshape(NW, NBLK, G, C)
    w_p = jnp.concatenate(
        [adj_edge_weight, jnp.zeros((pad,), jnp.float32)]).reshape(NW, NBLK, G, C)

    hi2 = _spmm(input, src_p, dst_p, w_p)

    scal = jnp.stack([jnp.float32(1) - alpha, alpha, theta])
    return _combine(scal, hi2, h0, weight)
